# baseline (speedup 1.0000x reference)
import functools

import jax
import jax.numpy as jnp
from jax import lax
from jax.experimental import pallas as pl
from jax.experimental.pallas import tpu as pltpu
from jax.experimental.pallas import tpu_sc as plsc

_T, _D, _E = 8192, 2048, 64
_NC, _NS, _L = 2, 16, 16
_NW = _NC * _NS
_ROWS_W = _T // _NW
_GROUPS_W = _ROWS_W // _L


def _logits_block(x_ref, w_ref, logits_ref, load_ref):
    x = x_ref[...]
    w = w_ref[...]
    logits = lax.dot_general(
        x, w, (((1,), (1,)), ((), ())),
        preferred_element_type=jnp.float32)
    logits_ref[...] = logits
    m = jnp.max(logits, axis=1, keepdims=True)
    ex = jnp.exp(logits - m)
    load_ref[...] = ex / jnp.sum(ex, axis=1, keepdims=True)


def _tc_stage(x, w_gate, block_t=512):
    t, d = x.shape
    e = w_gate.shape[0]
    return pl.pallas_call(
        _logits_block,
        grid=(t // block_t,),
        in_specs=[
            pl.BlockSpec((block_t, d), lambda i: (i, 0)),
            pl.BlockSpec((e, d), lambda i: (0, 0)),
        ],
        out_specs=[
            pl.BlockSpec((block_t, e), lambda i: (i, 0)),
            pl.BlockSpec((block_t, e), lambda i: (i, 0)),
        ],
        out_shape=[
            jax.ShapeDtypeStruct((t, e), jnp.float32),
            jax.ShapeDtypeStruct((t, e), jnp.float32),
        ],
    )(x, w_gate)


def _sc_gates_body(logits_hbm, gates_hbm, lv, gv):
    c = lax.axis_index("c")
    s = lax.axis_index("s")
    wid = s * _NC + c
    base = wid * (_ROWS_W * _E)

    pltpu.sync_copy(logits_hbm.at[pl.ds(base, _ROWS_W * _E)], lv)

    zeros = jnp.zeros((_L,), jnp.float32)

    def _zero(i, carry):
        gv[pl.ds(pl.multiple_of(i * _L, _L), _L)] = zeros
        return carry

    lax.fori_loop(0, _ROWS_W * _E // _L, _zero, 0)

    iota = lax.iota(jnp.int32, _L)
    minf = jnp.full((_L,), -jnp.inf, jnp.float32)
    izero = jnp.zeros((_L,), jnp.int32)

    def _group(g, carry):
        row0 = g * _L * _E
        colbase = row0 + iota * _E

        def _expert(e_i, st):
            m1, i1, m2, i2 = st
            v = plsc.load_gather(lv, [colbase + e_i])
            gt1 = v > m1
            gt2 = v > m2
            m2n = jnp.where(gt1, m1, jnp.where(gt2, v, m2))
            i2n = jnp.where(gt1, i1, jnp.where(gt2, e_i, i2))
            m1n = jnp.where(gt1, v, m1)
            i1n = jnp.where(gt1, e_i, i1)
            return m1n, i1n, m2n, i2n

        m1, i1, m2, i2 = lax.fori_loop(
            0, _E, _expert, (minf, izero, minf, izero))

        e2 = jnp.exp(m2 - m1)
        den = 1.0 + e2
        plsc.store_scatter(gv, [colbase + i1], 1.0 / den)
        plsc.store_scatter(gv, [colbase + i2], e2 / den)
        return carry

    lax.fori_loop(0, _GROUPS_W, _group, 0)

    pltpu.sync_copy(gv, gates_hbm.at[pl.ds(base, _ROWS_W * _E)])


@functools.partial(
    pl.kernel,
    out_type=jax.ShapeDtypeStruct((_T * _E,), jnp.float32),
    mesh=plsc.VectorSubcoreMesh(core_axis_name="c", subcore_axis_name="s"),
    scratch_types=[
        pltpu.VMEM((_ROWS_W * _E,), jnp.float32),
        pltpu.VMEM((_ROWS_W * _E,), jnp.float32),
    ],
    compiler_params=pltpu.CompilerParams(needs_layout_passes=False),
)
def _sc_gates(logits_hbm, gates_hbm, lv, gv):
    _sc_gates_body(logits_hbm, gates_hbm, lv, gv)


def kernel(x, W_gate, W_noise):
    del W_noise
    logits, load = _tc_stage(x, W_gate)
    gates = _sc_gates(logits.reshape(_T * _E)).reshape(_T, _E)
    return gates, load


# speedup vs baseline: 1.1239x; 1.1239x over previous
import functools

import jax
import jax.numpy as jnp
from jax import lax
from jax.experimental import pallas as pl
from jax.experimental.pallas import tpu as pltpu
from jax.experimental.pallas import tpu_sc as plsc

_T, _D, _E = 8192, 2048, 64
_NC, _NS, _L = 2, 16, 16
_NW = _NC * _NS
_ROWS_W = _T // _NW
_GROUPS_W = _ROWS_W // _L


def _logits_block(x_ref, w_ref, logits_ref, load_ref):
    x = x_ref[...]
    w = w_ref[...]
    logits = lax.dot_general(
        x, w, (((1,), (1,)), ((), ())),
        preferred_element_type=jnp.float32)
    logits_ref[...] = logits
    m = jnp.max(logits, axis=1, keepdims=True)
    ex = jnp.exp(logits - m)
    load_ref[...] = ex / jnp.sum(ex, axis=1, keepdims=True)


def _tc_stage(x, w_gate, block_t=512):
    t, d = x.shape
    e = w_gate.shape[0]
    return pl.pallas_call(
        _logits_block,
        grid=(t // block_t,),
        in_specs=[
            pl.BlockSpec((block_t, d), lambda i: (i, 0)),
            pl.BlockSpec((e, d), lambda i: (0, 0)),
        ],
        out_specs=[
            pl.BlockSpec((block_t, e), lambda i: (i, 0)),
            pl.BlockSpec((block_t, e), lambda i: (i, 0)),
        ],
        out_shape=[
            jax.ShapeDtypeStruct((t, e), jnp.float32),
            jax.ShapeDtypeStruct((t, e), jnp.float32),
        ],
    )(x, w_gate)


def _sc_gates_body(logits_hbm, gates_hbm, lv, gv):
    c = lax.axis_index("c")
    s = lax.axis_index("s")
    wid = s * _NC + c
    base = wid * (_ROWS_W * _E)

    pltpu.sync_copy(logits_hbm.at[pl.ds(base, _ROWS_W * _E)], lv)

    zeros = jnp.zeros((_L,), jnp.float32)
    iota = lax.iota(jnp.int32, _L)
    minf = jnp.full((_L,), -jnp.inf, jnp.float32)
    izero = jnp.zeros((_L,), jnp.int32)

    def _group(g, carry):
        row0 = g * (_L * _E)
        row0 = pl.multiple_of(row0, _L * _E)
        colbase = row0 + iota * _E

        for k in range(_L * _E // _L):
            gv[pl.ds(row0 + k * _L, _L)] = zeros

        m1, i1, m2, i2 = minf, izero, minf, izero
        for e_i in range(_E):
            v = plsc.load_gather(lv, [colbase + e_i])
            ev = jnp.full((_L,), e_i, jnp.int32)
            gt1 = v > m1
            gt2 = v > m2
            m2n = jnp.where(gt1, m1, jnp.where(gt2, v, m2))
            i2n = jnp.where(gt1, i1, jnp.where(gt2, ev, i2))
            m1 = jnp.where(gt1, v, m1)
            i1 = jnp.where(gt1, ev, i1)
            m2, i2 = m2n, i2n

        e2 = jnp.exp(m2 - m1)
        den = 1.0 + e2
        plsc.store_scatter(gv, [colbase + i1], 1.0 / den)
        plsc.store_scatter(gv, [colbase + i2], e2 / den)
        return carry

    lax.fori_loop(0, _GROUPS_W, _group, 0, unroll=False)

    pltpu.sync_copy(gv, gates_hbm.at[pl.ds(base, _ROWS_W * _E)])


@functools.partial(
    pl.kernel,
    out_type=jax.ShapeDtypeStruct((_T * _E,), jnp.float32),
    mesh=plsc.VectorSubcoreMesh(core_axis_name="c", subcore_axis_name="s"),
    scratch_types=[
        pltpu.VMEM((_ROWS_W * _E,), jnp.float32),
        pltpu.VMEM((_ROWS_W * _E,), jnp.float32),
    ],
    compiler_params=pltpu.CompilerParams(needs_layout_passes=False),
)
def _sc_gates(logits_hbm, gates_hbm, lv, gv):
    _sc_gates_body(logits_hbm, gates_hbm, lv, gv)


def kernel(x, W_gate, W_noise):
    del W_noise
    logits, load = _tc_stage(x, W_gate)
    gates = _sc_gates(logits.reshape(_T * _E)).reshape(_T, _E)
    return gates, load
